# Initial kernel scaffold; baseline (speedup 1.0000x reference)
import jax
import jax.numpy as jnp
from jax.experimental import pallas as pl


def kernel(token_seq, pos_mask, tok_emb, pos_emb, mlm_w, mlm_b, layer0_w_qkv, layer0_b_qkv, layer0_w_o, layer0_b_o, layer0_w_1, layer0_b_1, layer0_w_2, layer0_b_2, layer0_ln1_g, layer0_ln1_b, layer0_ln2_g, layer0_ln2_b, layer1_w_qkv, layer1_b_qkv, layer1_w_o, layer1_b_o, layer1_w_1, layer1_b_1, layer1_w_2, layer1_b_2, layer1_ln1_g, layer1_ln1_b, layer1_ln2_g, layer1_ln2_b):
    raise NotImplementedError("write your pallas kernel here")


# speedup vs baseline: 2.0490x; 2.0490x over previous
import functools

import jax
import jax.numpy as jnp
from jax import lax
from jax.experimental import pallas as pl
from jax.experimental.pallas import tpu as pltpu

_LN_EPS = 1e-6
_NEG_INF = -1e9
_HEADS = 12
_VMEM_LIMIT = 64 * 1024 * 1024


def _ln(x, g, b):
    mu = jnp.mean(x, axis=-1, keepdims=True)
    xc = x - mu
    var = jnp.mean(xc * xc, axis=-1, keepdims=True)
    return g * xc * lax.rsqrt(var + _LN_EPS) + b


def _tblock_kernel(x_ref, kb_ref, *refs, heads, seq, n_layers):
    out_ref = refs[-1]
    wr = refs[:-1]
    x = x_ref[...].astype(jnp.float32)
    r = x.shape[0]
    hid = x.shape[1]
    dh = hid // heads

    ri = lax.broadcasted_iota(jnp.int32, (r, r), 0) // seq
    ci = lax.broadcasted_iota(jnp.int32, (r, r), 1) // seq
    kb = kb_ref[0]
    bias = jnp.where(ri == ci, jnp.broadcast_to(kb, (r, r)), _NEG_INF)

    for li in range(n_layers):
        (g1, b1, wqkv, bqkv, wo, bo,
         g2, b2, w1, bb1, w2, bb2) = wr[12 * li:12 * (li + 1)]
        xn = _ln(x, g1[...], b1[...])
        qkv = (jnp.dot(xn.astype(jnp.bfloat16), wqkv[...],
                       preferred_element_type=jnp.float32)
               + bqkv[...]).astype(jnp.bfloat16)
        parts = []
        for hi in range(heads):
            sl = slice(hi * dh, (hi + 1) * dh)
            q = qkv[:, sl]
            k = qkv[:, hid + hi * dh:hid + (hi + 1) * dh]
            v = qkv[:, 2 * hid + hi * dh:2 * hid + (hi + 1) * dh]
            s = lax.dot_general(q, k, (((1,), (1,)), ((), ())),
                                preferred_element_type=jnp.float32)
            s = s + bias
            m = jnp.max(s, axis=-1, keepdims=True)
            p = jnp.exp(s - m)
            l = jnp.sum(p, axis=-1, keepdims=True)
            o = jnp.dot(p.astype(jnp.bfloat16), v,
                        preferred_element_type=jnp.float32)
            parts.append(o / l)
        ctx = jnp.concatenate(parts, axis=-1).astype(jnp.bfloat16)
        x = x + jnp.dot(ctx, wo[...],
                        preferred_element_type=jnp.float32) + bo[...]
        xn2 = _ln(x, g2[...], b2[...])
        hmid = jax.nn.gelu(
            jnp.dot(xn2.astype(jnp.bfloat16), w1[...],
                    preferred_element_type=jnp.float32) + bb1[...]
        ).astype(jnp.bfloat16)
        x = x + jnp.dot(hmid, w2[...],
                        preferred_element_type=jnp.float32) + bb2[...]
    out_ref[...] = x.astype(out_ref.dtype)


def _transformer_call(x2d, kb_g, layer_arrays, *, heads, seq, r):
    rows, hid = x2d.shape
    steps = rows // r
    specs = [
        pl.BlockSpec((r, hid), lambda i: (i, 0)),
        pl.BlockSpec((1, 1, r), lambda i: (i, 0, 0)),
    ]
    args = [x2d, kb_g.reshape(steps, 1, r)]
    for arr in layer_arrays:
        specs.append(pl.BlockSpec(arr.shape, lambda i: (0, 0)))
        args.append(arr)
    return pl.pallas_call(
        functools.partial(_tblock_kernel, heads=heads, seq=seq,
                          n_layers=len(layer_arrays) // 12),
        grid=(steps,),
        in_specs=specs,
        out_specs=pl.BlockSpec((r, hid), lambda i: (i, 0)),
        out_shape=jax.ShapeDtypeStruct((rows, hid), jnp.bfloat16),
        compiler_params=pltpu.CompilerParams(
            dimension_semantics=("parallel",),
            vmem_limit_bytes=_VMEM_LIMIT),
    )(*args)


def _lse_kernel(x_ref, w_ref, b_ref, o_ref, m_sc, l_sc, *, tv, vocab):
    j = pl.program_id(1)

    @pl.when(j == 0)
    def _():
        m_sc[...] = jnp.full_like(m_sc, -jnp.inf)
        l_sc[...] = jnp.zeros_like(l_sc)

    s = jnp.dot(x_ref[...], w_ref[...],
                preferred_element_type=jnp.float32) + b_ref[...]
    col = j * tv + lax.broadcasted_iota(jnp.int32, s.shape, 1)
    s = jnp.where(col < vocab, s, _NEG_INF)
    m_prev = m_sc[...]
    m_new = jnp.maximum(m_prev, jnp.max(s, axis=-1, keepdims=True))
    l_sc[...] = (l_sc[...] * jnp.exp(m_prev - m_new)
                 + jnp.sum(jnp.exp(s - m_new), axis=-1, keepdims=True))
    m_sc[...] = m_new

    @pl.when(j == pl.num_programs(1) - 1)
    def _():
        o_ref[...] = jnp.broadcast_to(m_sc[...] + jnp.log(l_sc[...]),
                                      o_ref.shape)


def _lse_call(x2d, w, b2d, *, tv, lse_w):
    rows, hid = x2d.shape
    vocab = w.shape[1]
    tm = rows // 2
    jn = pl.cdiv(vocab, tv)
    return pl.pallas_call(
        functools.partial(_lse_kernel, tv=tv, vocab=vocab),
        grid=(2, jn),
        in_specs=[
            pl.BlockSpec((tm, hid), lambda c, j: (c, 0)),
            pl.BlockSpec((hid, tv), lambda c, j: (0, j)),
            pl.BlockSpec((1, tv), lambda c, j: (0, j)),
        ],
        out_specs=pl.BlockSpec((tm, lse_w), lambda c, j: (c, 0)),
        out_shape=jax.ShapeDtypeStruct((rows, lse_w), jnp.float32),
        scratch_shapes=[pltpu.VMEM((tm, 1), jnp.float32),
                        pltpu.VMEM((tm, 1), jnp.float32)],
        compiler_params=pltpu.CompilerParams(
            dimension_semantics=("parallel", "arbitrary"),
            vmem_limit_bytes=_VMEM_LIMIT),
    )(x2d, w, b2d)


def _logits_kernel(x_ref, w_ref, b_ref, lse_ref, o_ref):
    s = jnp.dot(x_ref[...], w_ref[...],
                preferred_element_type=jnp.float32) + b_ref[...]
    o_ref[...] = s - lse_ref[...]


def _logits_call(x2d, w, b2d, lse, *, tv):
    rows, hid = x2d.shape
    vocab = w.shape[1]
    tm = rows // 2
    jn = pl.cdiv(vocab, tv)
    return pl.pallas_call(
        _logits_kernel,
        grid=(2, jn),
        in_specs=[
            pl.BlockSpec((tm, hid), lambda c, j: (c, 0)),
            pl.BlockSpec((hid, tv), lambda c, j: (0, j)),
            pl.BlockSpec((1, tv), lambda c, j: (0, j)),
            pl.BlockSpec((tm, tv), lambda c, j: (c, 0)),
        ],
        out_specs=pl.BlockSpec((tm, tv), lambda c, j: (c, j)),
        out_shape=jax.ShapeDtypeStruct((rows, vocab), jnp.float32),
        compiler_params=pltpu.CompilerParams(
            dimension_semantics=("parallel", "arbitrary"),
            vmem_limit_bytes=_VMEM_LIMIT),
    )(x2d, w, b2d, lse)


def kernel(token_seq, pos_mask, tok_emb, pos_emb, mlm_w, mlm_b,
           layer0_w_qkv, layer0_b_qkv, layer0_w_o, layer0_b_o,
           layer0_w_1, layer0_b_1, layer0_w_2, layer0_b_2,
           layer0_ln1_g, layer0_ln1_b, layer0_ln2_g, layer0_ln2_b,
           layer1_w_qkv, layer1_b_qkv, layer1_w_o, layer1_b_o,
           layer1_w_1, layer1_b_1, layer1_w_2, layer1_b_2,
           layer1_ln1_g, layer1_ln1_b, layer1_ln2_g, layer1_ln2_b):
    b, s = token_seq.shape
    hid = tok_emb.shape[1]
    rows = b * s

    x2d = (tok_emb[token_seq] + pos_emb[pos_mask]).astype(
        jnp.bfloat16).reshape(rows, hid)
    kb = jnp.where(token_seq > 0, 0.0, _NEG_INF).astype(jnp.float32)

    g = max(1, 512 // s)
    while b % g:
        g -= 1
    r = g * s

    def row(v):
        return v.reshape(1, -1)

    layer_arrays = []
    for lp in ((layer0_ln1_g, layer0_ln1_b, layer0_w_qkv, layer0_b_qkv,
                layer0_w_o, layer0_b_o, layer0_ln2_g, layer0_ln2_b,
                layer0_w_1, layer0_b_1, layer0_w_2, layer0_b_2),
               (layer1_ln1_g, layer1_ln1_b, layer1_w_qkv, layer1_b_qkv,
                layer1_w_o, layer1_b_o, layer1_ln2_g, layer1_ln2_b,
                layer1_w_1, layer1_b_1, layer1_w_2, layer1_b_2)):
        for a in lp:
            layer_arrays.append(a if a.ndim == 2 else row(a))

    x2d = _transformer_call(x2d, kb.reshape(rows // r, r), layer_arrays,
                            heads=_HEADS, seq=s, r=r)

    tv = 512
    b2d = row(mlm_b)
    lse = _lse_call(x2d, mlm_w, b2d, tv=tv, lse_w=tv)
    out = _logits_call(x2d, mlm_w, b2d, lse, tv=tv)
    return out.reshape(b, s, -1)
